# baseline (speedup 1.0000x reference)
import functools

import jax
import jax.numpy as jnp
from jax import lax
from jax.experimental import pallas as pl
from jax.experimental.pallas import tpu as pltpu
from jax.experimental.pallas import tpu_sc as plsc

_EPS = 1e-9
_L = 16


def _make_sc_call(total_rows: int, C: int):
    info = plsc.get_sparse_core_info()
    NC, NS = info.num_cores, info.num_subcores
    NW = NC * NS
    assert total_rows % NW == 0
    rows_per_w = total_rows // NW
    chunk = 480
    while rows_per_w % chunk:
        chunk //= 2
    n_chunks = rows_per_w // chunk
    n_groups = chunk // _L

    mesh = plsc.VectorSubcoreMesh(core_axis_name="c", subcore_axis_name="s")

    @functools.partial(
        pl.kernel,
        out_type=(
            jax.ShapeDtypeStruct((NW, _L), jnp.float32),
            jax.ShapeDtypeStruct((NW, _L), jnp.float32),
        ),
        mesh=mesh,
        compiler_params=pltpu.CompilerParams(needs_layout_passes=False),
        scratch_types=(
            pltpu.VMEM((chunk * C,), jnp.float32),
            pltpu.VMEM((chunk * 4,), jnp.float32),
            pltpu.VMEM((chunk * 4,), jnp.float32),
            pltpu.VMEM((chunk,), jnp.float32),
            pltpu.VMEM((_L,), jnp.float32),
            pltpu.VMEM((_L,), jnp.float32),
        ),
    )
    def sc_call(scores_hbm, pred_hbm, tgt_hbm, mask_hbm, loss_out, ts_out,
                scores_v, pred_v, tgt_v, mask_v, accl_v, acct_v):
        wid = lax.axis_index("s") * NC + lax.axis_index("c")
        base0 = wid * rows_per_w
        lane = lax.iota(jnp.int32, _L)
        rowC = lane * C
        row4 = lane * 4
        zero = jnp.zeros((_L,), jnp.float32)

        def chunk_body(i, carry):
            acc_l, acc_t = carry
            base = base0 + i * chunk
            pltpu.sync_copy(scores_hbm.at[pl.ds(base * C, chunk * C)], scores_v)
            pltpu.sync_copy(pred_hbm.at[pl.ds(base * 4, chunk * 4)], pred_v)
            pltpu.sync_copy(tgt_hbm.at[pl.ds(base * 4, chunk * 4)], tgt_v)
            pltpu.sync_copy(mask_hbm.at[pl.ds(base, chunk)], mask_v)

            def group_body(j, carry2):
                acc_l, acc_t = carry2
                idx = rowC + j * (_L * C)
                w = zero
                for _ in range(C):
                    w = w + plsc.load_gather(scores_v, [idx])
                    idx = idx + 1
                bidx = row4 + j * (_L * 4)
                px1 = plsc.load_gather(pred_v, [bidx])
                py1 = plsc.load_gather(pred_v, [bidx + 1])
                px2 = plsc.load_gather(pred_v, [bidx + 2])
                py2 = plsc.load_gather(pred_v, [bidx + 3])
                tx1 = plsc.load_gather(tgt_v, [bidx])
                ty1 = plsc.load_gather(tgt_v, [bidx + 1])
                tx2 = plsc.load_gather(tgt_v, [bidx + 2])
                ty2 = plsc.load_gather(tgt_v, [bidx + 3])
                iw = jnp.maximum(jnp.minimum(px2, tx2) - jnp.maximum(px1, tx1), 0.0)
                ih = jnp.maximum(jnp.minimum(py2, ty2) - jnp.maximum(py1, ty1), 0.0)
                inter = iw * ih
                area1 = jnp.maximum(px2 - px1, 0.0) * jnp.maximum(py2 - py1, 0.0)
                area2 = jnp.maximum(tx2 - tx1, 0.0) * jnp.maximum(ty2 - ty1, 0.0)
                union = area1 + area2 - inter + _EPS
                iou = inter / union
                cw = jnp.maximum(px2, tx2) - jnp.minimum(px1, tx1)
                ch = jnp.maximum(py2, ty2) - jnp.minimum(py1, ty1)
                c_area = cw * ch + _EPS
                giou = iou - (c_area - union) / c_area
                m = mask_v[pl.ds(j * _L, _L)]
                acc_l = acc_l + (1.0 - giou) * w * m
                acc_t = acc_t + w
                return acc_l, acc_t

            return lax.fori_loop(0, n_groups, group_body, (acc_l, acc_t))

        acc_l, acc_t = lax.fori_loop(0, n_chunks, chunk_body, (zero, zero))
        accl_v[...] = acc_l
        acct_v[...] = acc_t
        pltpu.sync_copy(accl_v, loss_out.at[wid])
        pltpu.sync_copy(acct_v, ts_out.at[wid])

    return sc_call


def kernel(pred_bboxes, target_bboxes, target_scores, mask_positive):
    B, N, C = target_scores.shape
    total_rows = B * N
    scores_flat = target_scores.reshape(-1)
    pred_flat = pred_bboxes.reshape(-1)
    tgt_flat = target_bboxes.reshape(-1)
    mask_flat = mask_positive.reshape(-1).astype(jnp.float32)
    loss_p, ts_p = _make_sc_call(total_rows, C)(
        scores_flat, pred_flat, tgt_flat, mask_flat)
    loss = loss_p.sum()
    ts = ts_p.sum()
    return jnp.where(ts > 1.0, loss / ts, loss)


# speedup vs baseline: 5.8630x; 5.8630x over previous
import functools

import jax
import jax.numpy as jnp
from jax import lax
from jax.experimental import pallas as pl
from jax.experimental.pallas import tpu as pltpu
from jax.experimental.pallas import tpu_sc as plsc

_EPS = 1e-9
_L = 16


def _make_sc_call(B: int, N: int, C: int):
    info = plsc.get_sparse_core_info()
    NC, NS = info.num_cores, info.num_subcores
    NW = NC * NS
    FULL = N // 128
    n_full = B * FULL
    assert n_full % NW == 0 and B % NW == 0
    fpw = n_full // NW
    tpw = B // NW

    mesh = plsc.VectorSubcoreMesh(core_axis_name="c", subcore_axis_name="s")

    @functools.partial(
        pl.kernel,
        out_type=(
            jax.ShapeDtypeStruct((NW, _L), jnp.float32),
            jax.ShapeDtypeStruct((NW, _L), jnp.float32),
        ),
        mesh=mesh,
        compiler_params=pltpu.CompilerParams(
            needs_layout_passes=False, use_tc_tiling_on_sc=True),
        scratch_types=(
            pltpu.VMEM((C, 128), jnp.float32),
            pltpu.VMEM((4, 128), jnp.float32),
            pltpu.VMEM((4, 128), jnp.float32),
            pltpu.VMEM((128,), jnp.float32),
            pltpu.VMEM((_L,), jnp.float32),
            pltpu.VMEM((_L,), jnp.float32),
        ),
    )
    def sc_call(ts_hbm, pb_hbm, tb_hbm, mk_hbm,
                tst_hbm, pbt_hbm, tbt_hbm, mkt_hbm,
                loss_out, ts_out, xv, pv, tv, mv, l_v, t_v):
        wid = lax.axis_index("s") * NC + lax.axis_index("c")

        def load_full(b, col):
            pltpu.sync_copy(ts_hbm.at[b, :, pl.ds(col, 128)], xv)
            pltpu.sync_copy(pb_hbm.at[b, :, pl.ds(col, 128)], pv)
            pltpu.sync_copy(tb_hbm.at[b, :, pl.ds(col, 128)], tv)
            pltpu.sync_copy(mk_hbm.at[b, pl.ds(col, 128)], mv)

        def load_tail(b):
            pltpu.sync_copy(tst_hbm.at[b], xv)
            pltpu.sync_copy(pbt_hbm.at[b], pv)
            pltpu.sync_copy(tbt_hbm.at[b], tv)
            pltpu.sync_copy(mkt_hbm.at[b], mv)

        def unit_sums():
            def row_body(r, accs):
                return tuple(
                    accs[g] + xv[r, pl.ds(g * _L, _L)] for g in range(8))

            zeros = tuple(jnp.zeros((_L,), jnp.float32) for _ in range(8))
            return lax.fori_loop(0, C, row_body, zeros)

        def unit_loss(ws, acc_l, acc_t):
            for g in range(8):
                s = g * _L
                w = ws[g]
                px1 = pv[0, pl.ds(s, _L)]
                py1 = pv[1, pl.ds(s, _L)]
                px2 = pv[2, pl.ds(s, _L)]
                py2 = pv[3, pl.ds(s, _L)]
                tx1 = tv[0, pl.ds(s, _L)]
                ty1 = tv[1, pl.ds(s, _L)]
                tx2 = tv[2, pl.ds(s, _L)]
                ty2 = tv[3, pl.ds(s, _L)]
                iw = jnp.maximum(jnp.minimum(px2, tx2) - jnp.maximum(px1, tx1), 0.0)
                ih = jnp.maximum(jnp.minimum(py2, ty2) - jnp.maximum(py1, ty1), 0.0)
                inter = iw * ih
                area1 = jnp.maximum(px2 - px1, 0.0) * jnp.maximum(py2 - py1, 0.0)
                area2 = jnp.maximum(tx2 - tx1, 0.0) * jnp.maximum(ty2 - ty1, 0.0)
                union = area1 + area2 - inter + _EPS
                iou = inter / union
                cw = jnp.maximum(px2, tx2) - jnp.minimum(px1, tx1)
                ch = jnp.maximum(py2, ty2) - jnp.minimum(py1, ty1)
                c_area = cw * ch + _EPS
                giou = iou - (c_area - union) / c_area
                m = mv[pl.ds(s, _L)]
                acc_l = acc_l + (1.0 - giou) * w * m
                acc_t = acc_t + w
            return acc_l, acc_t

        def full_body(i, carry):
            acc_l, acc_t = carry
            u = wid * fpw + i
            b = u // FULL
            k = u - b * FULL
            load_full(b, pl.multiple_of(k * 128, 128))
            ws = unit_sums()
            return unit_loss(ws, acc_l, acc_t)

        def tail_body(i, carry):
            acc_l, acc_t = carry
            load_tail(wid * tpw + i)
            ws = unit_sums()
            return unit_loss(ws, acc_l, acc_t)

        zero = jnp.zeros((_L,), jnp.float32)
        acc = lax.fori_loop(0, fpw, full_body, (zero, zero))
        acc_l, acc_t = lax.fori_loop(0, tpw, tail_body, acc)
        l_v[...] = acc_l
        t_v[...] = acc_t
        pltpu.sync_copy(l_v, loss_out.at[wid])
        pltpu.sync_copy(t_v, ts_out.at[wid])

    return sc_call


def kernel(pred_bboxes, target_bboxes, target_scores, mask_positive):
    B, N, C = target_scores.shape
    FULL = N // 128
    ncols = FULL * 128
    pad = (N - ncols, 128 - (N - ncols))
    ts_t = target_scores.transpose(0, 2, 1)
    pb_t = pred_bboxes.transpose(0, 2, 1)
    tb_t = target_bboxes.transpose(0, 2, 1)
    mask_f = mask_positive.astype(jnp.float32)

    def pad_tail(x):
        tail = x[..., ncols:]
        return jnp.pad(tail, [(0, 0)] * (x.ndim - 1) + [(0, pad[1])])

    loss_p, ts_p = _make_sc_call(B, N, C)(
        ts_t, pb_t, tb_t, mask_f,
        pad_tail(ts_t), pad_tail(pb_t), pad_tail(tb_t), pad_tail(mask_f))
    loss = loss_p.sum()
    ts = ts_p.sum()
    return jnp.where(ts > 1.0, loss / ts, loss)


# speedup vs baseline: 17.8167x; 3.0388x over previous
import functools

import jax
import jax.numpy as jnp
from jax import lax
from jax.experimental import pallas as pl
from jax.experimental.pallas import tpu as pltpu
from jax.experimental.pallas import tpu_sc as plsc

_EPS = 1e-9
_L = 16


def _make_sc_call(B: int, N: int, C: int):
    info = plsc.get_sparse_core_info()
    NC, NS = info.num_cores, info.num_subcores
    NW = NC * NS
    FULL = N // 128
    n_full = B * FULL
    assert n_full % NW == 0 and B % NW == 0
    fpw = n_full // NW
    assert fpw % 2 == 0
    tpw = B // NW

    mesh = plsc.VectorSubcoreMesh(core_axis_name="c", subcore_axis_name="s")

    @functools.partial(
        pl.kernel,
        out_type=(
            jax.ShapeDtypeStruct((NW, _L), jnp.float32),
            jax.ShapeDtypeStruct((NW, _L), jnp.float32),
        ),
        mesh=mesh,
        compiler_params=pltpu.CompilerParams(
            needs_layout_passes=False, use_tc_tiling_on_sc=True),
        scratch_types=(
            pltpu.VMEM((C, 128), jnp.float32),
            pltpu.VMEM((C, 128), jnp.float32),
            pltpu.VMEM((4, 128), jnp.float32),
            pltpu.VMEM((4, 128), jnp.float32),
            pltpu.VMEM((4, 128), jnp.float32),
            pltpu.VMEM((4, 128), jnp.float32),
            pltpu.VMEM((128,), jnp.float32),
            pltpu.VMEM((128,), jnp.float32),
            pltpu.VMEM((_L,), jnp.float32),
            pltpu.VMEM((_L,), jnp.float32),
            pltpu.SemaphoreType.DMA,
            pltpu.SemaphoreType.DMA,
        ),
    )
    def sc_call(ts_hbm, pb_hbm, tb_hbm, mk_hbm,
                tst_hbm, pbt_hbm, tbt_hbm, mkt_hbm,
                loss_out, ts_out,
                xv0, xv1, pv0, pv1, tv0, tv1, mv0, mv1,
                l_v, t_v, sem0, sem1):
        wid = lax.axis_index("s") * NC + lax.axis_index("c")
        slots = ((xv0, pv0, tv0, mv0, sem0), (xv1, pv1, tv1, mv1, sem1))

        def unit_refs(u):
            b = u // FULL
            col = pl.multiple_of((u - b * FULL) * 128, 128)
            return (ts_hbm.at[b, :, pl.ds(col, 128)],
                    pb_hbm.at[b, :, pl.ds(col, 128)],
                    tb_hbm.at[b, :, pl.ds(col, 128)],
                    mk_hbm.at[b, pl.ds(col, 128)])

        def start(u, slot):
            xv, pv, tv, mv, sem = slots[slot]
            s_ts, s_pb, s_tb, s_mk = unit_refs(u)
            pltpu.async_copy(s_ts, xv, sem)
            pltpu.async_copy(s_pb, pv, sem)
            pltpu.async_copy(s_tb, tv, sem)
            pltpu.async_copy(s_mk, mv, sem)

        def wait(u, slot):
            xv, pv, tv, mv, sem = slots[slot]
            s_ts, s_pb, s_tb, s_mk = unit_refs(u)
            pltpu.make_async_copy(s_ts, xv, sem).wait()
            pltpu.make_async_copy(s_pb, pv, sem).wait()
            pltpu.make_async_copy(s_tb, tv, sem).wait()
            pltpu.make_async_copy(s_mk, mv, sem).wait()

        def unit_sums(xv):
            def row_body(r, accs):
                return tuple(
                    accs[g] + xv[r, pl.ds(g * _L, _L)] for g in range(8))

            zeros = tuple(jnp.zeros((_L,), jnp.float32) for _ in range(8))
            return lax.fori_loop(0, C, row_body, zeros, unroll=4)

        def unit_loss(slot_refs, ws, acc_l, acc_t):
            _, pv, tv, mv, _ = slot_refs
            for g in range(8):
                s = g * _L
                w = ws[g]
                px1 = pv[0, pl.ds(s, _L)]
                py1 = pv[1, pl.ds(s, _L)]
                px2 = pv[2, pl.ds(s, _L)]
                py2 = pv[3, pl.ds(s, _L)]
                tx1 = tv[0, pl.ds(s, _L)]
                ty1 = tv[1, pl.ds(s, _L)]
                tx2 = tv[2, pl.ds(s, _L)]
                ty2 = tv[3, pl.ds(s, _L)]
                iw = jnp.maximum(jnp.minimum(px2, tx2) - jnp.maximum(px1, tx1), 0.0)
                ih = jnp.maximum(jnp.minimum(py2, ty2) - jnp.maximum(py1, ty1), 0.0)
                inter = iw * ih
                area1 = jnp.maximum(px2 - px1, 0.0) * jnp.maximum(py2 - py1, 0.0)
                area2 = jnp.maximum(tx2 - tx1, 0.0) * jnp.maximum(ty2 - ty1, 0.0)
                union = area1 + area2 - inter + _EPS
                iou = inter / union
                cw = jnp.maximum(px2, tx2) - jnp.minimum(px1, tx1)
                ch = jnp.maximum(py2, ty2) - jnp.minimum(py1, ty1)
                c_area = cw * ch + _EPS
                giou = iou - (c_area - union) / c_area
                m = mv[pl.ds(s, _L)]
                acc_l = acc_l + (1.0 - giou) * w * m
                acc_t = acc_t + w
            return acc_l, acc_t

        u_base = wid * fpw
        start(u_base, 0)
        start(u_base + 1, 1)

        def pair_body(j, carry):
            acc_l, acc_t = carry
            u = u_base + 2 * j
            for slot in (0, 1):
                uu = u + slot
                wait(uu, slot)
                ws = unit_sums(slots[slot][0])

                @pl.when(2 * j + slot + 2 < fpw)
                def _():
                    start(uu + 2, slot)

                acc_l, acc_t = unit_loss(slots[slot], ws, acc_l, acc_t)
            return acc_l, acc_t

        zero = jnp.zeros((_L,), jnp.float32)
        acc = lax.fori_loop(0, fpw // 2, pair_body, (zero, zero))

        def tail_body(i, carry):
            acc_l, acc_t = carry
            b = wid * tpw + i
            xv, pv, tv, mv, _ = slots[0]
            pltpu.sync_copy(tst_hbm.at[b], xv)
            pltpu.sync_copy(pbt_hbm.at[b], pv)
            pltpu.sync_copy(tbt_hbm.at[b], tv)
            pltpu.sync_copy(mkt_hbm.at[b], mv)
            ws = unit_sums(xv)
            return unit_loss(slots[0], ws, acc_l, acc_t)

        acc_l, acc_t = lax.fori_loop(0, tpw, tail_body, acc)
        l_v[...] = acc_l
        t_v[...] = acc_t
        pltpu.sync_copy(l_v, loss_out.at[wid])
        pltpu.sync_copy(t_v, ts_out.at[wid])

    return sc_call


def kernel(pred_bboxes, target_bboxes, target_scores, mask_positive):
    B, N, C = target_scores.shape
    FULL = N // 128
    ncols = FULL * 128
    ts_t = target_scores.transpose(0, 2, 1)
    pb_t = pred_bboxes.transpose(0, 2, 1)
    tb_t = target_bboxes.transpose(0, 2, 1)
    mask_f = mask_positive.astype(jnp.float32)

    def pad_tail(x):
        tail = x[..., ncols:]
        return jnp.pad(tail, [(0, 0)] * (x.ndim - 1) + [(0, 128 - (N - ncols))])

    loss_p, ts_p = _make_sc_call(B, N, C)(
        ts_t, pb_t, tb_t, mask_f,
        pad_tail(ts_t), pad_tail(pb_t), pad_tail(tb_t), pad_tail(mask_f))
    loss = loss_p.sum()
    ts = ts_p.sum()
    return jnp.where(ts > 1.0, loss / ts, loss)
